# Initial kernel scaffold; baseline (speedup 1.0000x reference)
import jax
import jax.numpy as jnp
from jax.experimental import pallas as pl


def kernel(rec_x, rec_f, rec_vec, rec_edge_index, rec_edge_w, lig_x, lig_f, lig_edge_index, lig_edge_w, rWm0, rWa0, rWh0, rWm1, rWa1, rWh1, rWo, lWm0, lWa0, lWh0, lWm1, lWa1, lWh1, lWo, cWm0, cWa0, cWh0, cWm1, cWa1, cWh1, cWmf, cWdf):
    raise NotImplementedError("write your pallas kernel here")


# speedup vs baseline: 14.5962x; 14.5962x over previous
import functools

import jax
import jax.numpy as jnp
from jax.experimental import pallas as pl
from jax.experimental.pallas import tpu as pltpu

F32 = jnp.float32


BF16 = jnp.bfloat16


def _onehot_t(idx_row, n, te):
    iota = jax.lax.broadcasted_iota(jnp.int32, (n, te), 0)
    return (iota == jnp.broadcast_to(idx_row, (n, te))).astype(BF16)


def _split(v):
    hi = v.astype(BF16)
    lo = (v - hi.astype(F32)).astype(BF16)
    return hi, lo


def _gather_pair(ot, hi, lo):
    dn = (((0,), (0,)), ((), ()))
    return (jax.lax.dot_general(ot, hi, dn, preferred_element_type=F32)
            + jax.lax.dot_general(ot, lo, dn, preferred_element_type=F32))


def _gather(ot, a):
    hi, lo = _split(a)
    return _gather_pair(ot, hi, lo)


def _scat(ot, v):
    hi, lo = _split(v)
    dn = (((1,), (0,)), ((), ()))
    return (jax.lax.dot_general(ot, hi, dn, preferred_element_type=F32)
            + jax.lax.dot_general(ot, lo, dn, preferred_element_type=F32))


def _make_mp_kernel(n, nt, te, vnorm, wo):
    def body(*refs):
        it = iter(refs)
        h = next(it)[...]
        vec = next(it)[...] if vnorm else None
        x = next(it)[...]
        src_ref = next(it)
        dst_ref = next(it)
        ew_ref = next(it)
        wms = next(it)[...]
        wmd = next(it)[...]
        wme = next(it)[...]
        wmr = next(it)[...]
        wat = next(it)[...]
        whh = next(it)[...]
        wha = next(it)[...]
        wo_w = next(it)[...] if wo else None
        out_ref = next(it)

        if vnorm:
            vn = jnp.sqrt(jnp.sum(vec * vec, axis=1, keepdims=True) + 1e-12)
            h = jnp.concatenate([h, vn], axis=1)
        hb = h.astype(BF16)
        a = jnp.dot(hb, wms.astype(BF16), preferred_element_type=F32)
        b = jnp.dot(hb, wmd.astype(BF16), preferred_element_type=F32)
        hw = jnp.dot(hb, whh.astype(BF16), preferred_element_type=F32)
        c = a.shape[1]
        wmrb = wmr.astype(BF16).astype(F32)
        watb = wat.astype(BF16).astype(F32)
        ahi, alo = _split(a)
        bhi, blo = _split(b)
        xhi, xlo = _split(x)

        def tile(t, carry):
            num, den = carry
            srow = src_ref[pl.ds(t, 1), :]
            drow = dst_ref[pl.ds(t, 1), :]
            ots = _onehot_t(srow, n, te)
            otd = _onehot_t(drow, n, te)
            xs = _gather_pair(ots, xhi, xlo)
            xd = _gather_pair(otd, xhi, xlo)
            d = xd - xs
            r = jnp.sqrt(jnp.sum(d * d, axis=1, keepdims=True) + 1e-12)
            ewt = ew_ref[pl.ds(t * te, te), :]
            p = jnp.dot(ewt.astype(BF16), wme.astype(BF16),
                        preferred_element_type=F32)
            rb = r.astype(BF16).astype(F32)
            m = _gather_pair(ots, ahi, alo) + _gather_pair(otd, bhi, blo) \
                + p + rb * wmrb
            m = jnp.maximum(m, 0.0)
            mb = m.astype(BF16).astype(F32)
            tl = jnp.exp(jnp.sum(mb * watb, axis=1, keepdims=True))
            num = num + _scat(otd, tl * m)
            den = den + _scat(otd, tl)
            return num, den

        num0 = jnp.zeros((n, c), F32)
        den0 = jnp.zeros((n, 1), F32)
        num, den = jax.lax.fori_loop(0, nt, tile, (num0, den0))
        agg = num / (den + 1e-9)
        hnew = jnp.maximum(
            hw + jnp.dot(agg.astype(BF16), wha.astype(BF16),
                         preferred_element_type=F32), 0.0)
        if wo:
            hnew = jnp.dot(hnew.astype(BF16), wo_w.astype(BF16),
                           preferred_element_type=F32)
        out_ref[...] = hnew

    return body


def _mp_layer(h, vec, x, src2, dst2, ew, wms, wmd, wme, wmr, wat, whh, wha,
              wo_w):
    n = h.shape[0]
    nt, te = src2.shape
    args = [h] + ([vec] if vec is not None else []) + [x, src2, dst2, ew,
            wms, wmd, wme, wmr, wat, whh, wha] + (
                [wo_w] if wo_w is not None else [])
    out_c = wo_w.shape[1] if wo_w is not None else wha.shape[1]
    body = _make_mp_kernel(n, nt, te, vec is not None, wo_w is not None)
    return pl.pallas_call(
        body,
        out_shape=jax.ShapeDtypeStruct((n, out_c), F32),
    )(*args)


NREC = 512
NLIG = 64
NC = NREC + NLIG


def _make_cross_kernel(nt_rec, te):
    def body(hc_ref, x_ref, rsrc_ref, rdst_ref, lsrc_ref, ldst_ref,
             wms_ref, wmd_ref, trows_ref, wmr_ref, wat_ref, whh_ref,
             wha_ref, out_ref, a_ref, b_ref, numl_ref, denl_ref):
        hc = hc_ref[...]
        x = x_ref[...]
        wms = wms_ref[...]
        wmd = wmd_ref[...]
        trows = trows_ref[...]
        wmr = wmr_ref[...]
        wat = wat_ref[...]
        whh = whh_ref[...]
        wha = wha_ref[...]

        hcb = hc.astype(BF16)
        a = jnp.dot(hcb, wms.astype(BF16), preferred_element_type=F32)
        b = jnp.dot(hcb, wmd.astype(BF16), preferred_element_type=F32)
        a_ref[...] = a
        b_ref[...] = b
        hw = jnp.dot(hcb, whh.astype(BF16), preferred_element_type=F32)
        c = a.shape[1]
        trowsb = trows.astype(BF16).astype(F32)
        t0 = trowsb[0:1, :]
        t1 = trowsb[1:2, :]
        t2 = trowsb[2:3, :]
        wmrb = wmr.astype(BF16).astype(F32)
        watb = wat.astype(BF16).astype(F32)

        ahi, alo = _split(a)
        bhi, blo = _split(b)
        xhi, xlo = _split(x)

        def sp_tile(srow, drow, tvec, num, den):
            ots = _onehot_t(srow, NC, srow.shape[1])
            otd = _onehot_t(drow, NC, drow.shape[1])
            xs = _gather_pair(ots, xhi, xlo)
            xd = _gather_pair(otd, xhi, xlo)
            d = xd - xs
            r = jnp.sqrt(jnp.sum(d * d, axis=1, keepdims=True) + 1e-12)
            rb = r.astype(BF16).astype(F32)
            m = _gather_pair(ots, ahi, alo) + _gather_pair(otd, bhi, blo) \
                + tvec + rb * wmrb
            m = jnp.maximum(m, 0.0)
            mb = m.astype(BF16).astype(F32)
            tl = jnp.exp(jnp.sum(mb * watb, axis=1, keepdims=True))
            return num + _scat(otd, tl * m), den + _scat(otd, tl)

        def rec_tile(t, carry):
            num, den = carry
            return sp_tile(rsrc_ref[pl.ds(t, 1), :], rdst_ref[pl.ds(t, 1), :],
                           t0, num, den)

        num0 = jnp.zeros((NC, c), F32)
        den0 = jnp.zeros((NC, 1), F32)
        num, den = jax.lax.fori_loop(0, nt_rec, rec_tile, (num0, den0))
        num, den = sp_tile(lsrc_ref[0:1, :], ldst_ref[0:1, :], t1, num, den)

        ar = a[0:NREC, :]
        br = b[0:NREC, :]
        xr = x[0:NREC, :]

        def dj(j, carry):
            num_r, den_r = carry
            xj = x_ref[pl.ds(NREC + j, 1), :]
            dcol = xj - xr
            r1 = jnp.sqrt(jnp.sum(dcol * dcol, axis=1, keepdims=True) + 1e-12)
            aj = a_ref[pl.ds(NREC + j, 1), :]
            bj = b_ref[pl.ds(NREC + j, 1), :]
            r1b = r1.astype(BF16).astype(F32)
            m1 = jnp.maximum(ar + bj + t2 + r1b * wmrb, 0.0)
            m1b = m1.astype(BF16).astype(F32)
            tl1 = jnp.exp(jnp.sum(m1b * watb, axis=1, keepdims=True))
            numl_ref[pl.ds(j, 1), :] = jnp.sum(tl1 * m1, axis=0, keepdims=True)
            denl_ref[pl.ds(j, 1), :] = jnp.sum(tl1, axis=0, keepdims=True)
            m2 = jnp.maximum(aj + br + t2 + r1b * wmrb, 0.0)
            m2b = m2.astype(BF16).astype(F32)
            tl2 = jnp.exp(jnp.sum(m2b * watb, axis=1, keepdims=True))
            num_r = num_r + tl2 * m2
            den_r = den_r + tl2
            return num_r, den_r

        numr0 = jnp.zeros((NREC, c), F32)
        denr0 = jnp.zeros((NREC, 1), F32)
        num_r, den_r = jax.lax.fori_loop(0, NLIG, dj, (numr0, denr0))
        num = num + jnp.concatenate([num_r, numl_ref[...]], axis=0)
        den = den + jnp.concatenate([den_r, denl_ref[...]], axis=0)

        agg = num / (den + 1e-9)
        out_ref[...] = jnp.maximum(
            hw + jnp.dot(agg.astype(BF16), wha.astype(BF16),
                         preferred_element_type=F32), 0.0)

    return body


def _cross_layer(hc, x, rsrc2, rdst2, lsrc2, ldst2, wms, wmd, trows, wmr,
                 wat, whh, wha):
    nt_rec, te = rsrc2.shape
    c = wha.shape[1]
    body = _make_cross_kernel(nt_rec, te)
    return pl.pallas_call(
        body,
        out_shape=jax.ShapeDtypeStruct((NC, c), F32),
        scratch_shapes=[
            pltpu.VMEM((NC, c), F32),
            pltpu.VMEM((NC, c), F32),
            pltpu.VMEM((NLIG, c), F32),
            pltpu.VMEM((NLIG, 1), F32),
        ],
    )(hc, x, rsrc2, rdst2, lsrc2, ldst2, wms, wmd, trows, wmr, wat, whh,
      wha)


def _make_final_kernel(te):
    def body(hc_ref, x_ref, lsrc_ref, ldst_ref, wfs_ref, wfd_ref, trows_ref,
             wfr_ref, wdft_ref, out_ref, b_ref, upd_ref):
        hc = hc_ref[...]
        x = x_ref[...]
        wfs = wfs_ref[...]
        wfd = wfd_ref[...]
        trows = trows_ref[...]
        wfr = wfr_ref[...]
        wdft = wdft_ref[...]

        hcb = hc.astype(BF16)
        a = jnp.dot(hcb, wfs.astype(BF16), preferred_element_type=F32)
        b = jnp.dot(hcb, wfd.astype(BF16), preferred_element_type=F32)
        b_ref[...] = b
        c = a.shape[1]
        trowsb = trows.astype(BF16).astype(F32)
        t1 = trowsb[1:2, :]
        t2 = trowsb[2:3, :]
        wfrb = wfr.astype(BF16).astype(F32)
        wdftb = wdft.astype(BF16).astype(F32)

        srow = lsrc_ref[0:1, :]
        drow = ldst_ref[0:1, :]
        ots = _onehot_t(srow, NC, te)
        otd = _onehot_t(drow, NC, te)
        xs = _gather(ots, x)
        xd = _gather(otd, x)
        d = xd - xs
        r = jnp.sqrt(jnp.sum(d * d, axis=1, keepdims=True) + 1e-12)
        rb = r.astype(BF16).astype(F32)
        f = jnp.maximum(
            _gather(ots, a) + _gather(otd, b) + t1 + rb * wfrb, 0.0)
        fb = f.astype(BF16).astype(F32)
        s = jnp.sum(fb * wdftb, axis=1, keepdims=True)
        su = s * (d / (r + 1e-9))
        otd64 = _onehot_t(drow - NREC, NLIG, te)
        upd_ref[...] = _scat(otd64, su)

        ar = a[0:NREC, :]
        xr = x[0:NREC, :]

        def dj(j, carry):
            xj = x_ref[pl.ds(NREC + j, 1), :]
            dcol = xj - xr
            r1 = jnp.sqrt(jnp.sum(dcol * dcol, axis=1, keepdims=True) + 1e-12)
            bj = b_ref[pl.ds(NREC + j, 1), :]
            r1b = r1.astype(BF16).astype(F32)
            f1 = jnp.maximum(ar + bj + t2 + r1b * wfrb, 0.0)
            f1b = f1.astype(BF16).astype(F32)
            s1 = jnp.sum(f1b * wdftb, axis=1, keepdims=True)
            u1 = dcol / (r1 + 1e-9)
            contrib = jnp.sum(s1 * u1, axis=0, keepdims=True)
            upd_ref[pl.ds(j, 1), :] = upd_ref[pl.ds(j, 1), :] + contrib
            return carry

        jax.lax.fori_loop(0, NLIG, dj, 0)
        out_ref[...] = x[NREC:NC, :] + upd_ref[...]

    return body


def _final_layer(hc, x, lsrc2, ldst2, wfs, wfd, trows, wfr, wdft):
    te = lsrc2.shape[1]
    return pl.pallas_call(
        _make_final_kernel(te),
        out_shape=jax.ShapeDtypeStruct((NLIG, 3), F32),
        scratch_shapes=[
            pltpu.VMEM((NC, wfd.shape[1]), F32),
            pltpu.VMEM((NLIG, 3), F32),
        ],
    )(hc, x, lsrc2, ldst2, wfs, wfd, trows, wfr, wdft)


def kernel(rec_x, rec_f, rec_vec, rec_edge_index, rec_edge_w, lig_x, lig_f,
           lig_edge_index, lig_edge_w, rWm0, rWa0, rWh0, rWm1, rWa1, rWh1,
           rWo, lWm0, lWa0, lWh0, lWm1, lWa1, lWh1, lWo, cWm0, cWa0, cWh0,
           cWm1, cWa1, cWh1, cWmf, cWdf):
    te = 1024
    rs = rec_edge_index[0].reshape(-1, te)
    rd = rec_edge_index[1].reshape(-1, te)
    ls = lig_edge_index[0].reshape(1, -1)
    ld = lig_edge_index[1].reshape(1, -1)
    vec = rec_vec[:, 0, :]

    h = _mp_layer(rec_f, vec, rec_x, rs, rd, rec_edge_w,
                  rWm0[0:129], rWm0[129:258], rWm0[258:262], rWm0[262:263],
                  rWa0.T, rWh0[0:129], rWh0[129:257], None)
    h_rec = _mp_layer(h, None, rec_x, rs, rd, rec_edge_w,
                      rWm1[0:128], rWm1[128:256], rWm1[256:260],
                      rWm1[260:261], rWa1.T, rWh1[0:128], rWh1[128:256], rWo)

    h = _mp_layer(lig_f, None, lig_x, ls, ld, lig_edge_w,
                  lWm0[0:64], lWm0[64:128], lWm0[128:132], lWm0[132:133],
                  lWa0.T, lWh0[0:64], lWh0[64:128], None)
    h_lig = _mp_layer(h, None, lig_x, ls, ld, lig_edge_w,
                      lWm1[0:64], lWm1[64:128], lWm1[128:132], lWm1[132:133],
                      lWa1.T, lWh1[0:64], lWh1[64:128], lWo)

    hc = jnp.concatenate([h_rec, h_lig], axis=0)
    x = jnp.concatenate([rec_x, lig_x], axis=0)
    lss = ls + NREC
    lds = ld + NREC
    hc = _cross_layer(hc, x, rs, rd, lss, lds,
                      cWm0[0:128], cWm0[128:256], cWm0[256:259],
                      cWm0[259:260], cWa0.T, cWh0[0:128], cWh0[128:256])
    hc = _cross_layer(hc, x, rs, rd, lss, lds,
                      cWm1[0:128], cWm1[128:256], cWm1[256:259],
                      cWm1[259:260], cWa1.T, cWh1[0:128], cWh1[128:256])

    lig_new = _final_layer(hc, x, lss, lds,
                           cWmf[0:128], cWmf[128:256], cWmf[256:259],
                           cWmf[259:260], cWdf.T)
    return jnp.stack([lig_x, lig_new])[None, :]


# speedup vs baseline: 15.0919x; 1.0340x over previous
import functools

import jax
import jax.numpy as jnp
from jax.experimental import pallas as pl
from jax.experimental.pallas import tpu as pltpu

F32 = jnp.float32


BF16 = jnp.bfloat16


def _onehot_t(idx_row, n, te):
    iota = jax.lax.broadcasted_iota(jnp.int32, (n, te), 0)
    return (iota == jnp.broadcast_to(idx_row, (n, te))).astype(BF16)


def _split(v):
    hi = v.astype(BF16)
    lo = (v - hi.astype(F32)).astype(BF16)
    return hi, lo


def _gather_pair(ot, hi, lo):
    dn = (((0,), (0,)), ((), ()))
    return (jax.lax.dot_general(ot, hi, dn, preferred_element_type=F32)
            + jax.lax.dot_general(ot, lo, dn, preferred_element_type=F32))


def _gather(ot, a):
    hi, lo = _split(a)
    return _gather_pair(ot, hi, lo)


def _scat(ot, v):
    hi, lo = _split(v)
    dn = (((1,), (0,)), ((), ()))
    return (jax.lax.dot_general(ot, hi, dn, preferred_element_type=F32)
            + jax.lax.dot_general(ot, lo, dn, preferred_element_type=F32))


def _make_mp_kernel(n, nt, te, vnorm, wo):
    def body(*refs):
        it = iter(refs)
        h = next(it)[...]
        vec = next(it)[...] if vnorm else None
        x = next(it)[...]
        src_ref = next(it)
        dst_ref = next(it)
        ew_ref = next(it)
        wms = next(it)[...]
        wmd = next(it)[...]
        wme = next(it)[...]
        wmr = next(it)[...]
        wat = next(it)[...]
        whh = next(it)[...]
        wha = next(it)[...]
        wo_w = next(it)[...] if wo else None
        out_ref = next(it)

        if vnorm:
            vn = jnp.sqrt(jnp.sum(vec * vec, axis=1, keepdims=True) + 1e-12)
            h = jnp.concatenate([h, vn], axis=1)
        hb = h.astype(BF16)
        a = jnp.dot(hb, wms.astype(BF16), preferred_element_type=F32)
        b = jnp.dot(hb, wmd.astype(BF16), preferred_element_type=F32)
        hw = jnp.dot(hb, whh.astype(BF16), preferred_element_type=F32)
        c = a.shape[1]
        wmrb = wmr.astype(BF16).astype(F32)
        watb = wat.astype(BF16).astype(F32)
        ahi, alo = _split(a)
        bhi, blo = _split(b)
        xhi, xlo = _split(x)

        def tile(t, carry):
            num, den = carry
            srow = src_ref[pl.ds(t, 1), :]
            drow = dst_ref[pl.ds(t, 1), :]
            ots = _onehot_t(srow, n, te)
            otd = _onehot_t(drow, n, te)
            xs = _gather_pair(ots, xhi, xlo)
            xd = _gather_pair(otd, xhi, xlo)
            d = xd - xs
            r = jnp.sqrt(jnp.sum(d * d, axis=1, keepdims=True) + 1e-12)
            ewt = ew_ref[pl.ds(t * te, te), :]
            p = jnp.dot(ewt.astype(BF16), wme.astype(BF16),
                        preferred_element_type=F32)
            rb = r.astype(BF16).astype(F32)
            m = _gather_pair(ots, ahi, alo) + _gather_pair(otd, bhi, blo) \
                + p + rb * wmrb
            m = jnp.maximum(m, 0.0)
            mb = m.astype(BF16).astype(F32)
            tl = jnp.exp(jnp.sum(mb * watb, axis=1, keepdims=True))
            num = num + _scat(otd, tl * m)
            den = den + _scat(otd, tl)
            return num, den

        num0 = jnp.zeros((n, c), F32)
        den0 = jnp.zeros((n, 1), F32)
        num, den = jax.lax.fori_loop(0, nt, tile, (num0, den0))
        agg = num / (den + 1e-9)
        hnew = jnp.maximum(
            hw + jnp.dot(agg.astype(BF16), wha.astype(BF16),
                         preferred_element_type=F32), 0.0)
        if wo:
            hnew = jnp.dot(hnew.astype(BF16), wo_w.astype(BF16),
                           preferred_element_type=F32)
        out_ref[...] = hnew

    return body


def _mp_layer(h, vec, x, src2, dst2, ew, wms, wmd, wme, wmr, wat, whh, wha,
              wo_w):
    n = h.shape[0]
    nt, te = src2.shape
    args = [h] + ([vec] if vec is not None else []) + [x, src2, dst2, ew,
            wms, wmd, wme, wmr, wat, whh, wha] + (
                [wo_w] if wo_w is not None else [])
    out_c = wo_w.shape[1] if wo_w is not None else wha.shape[1]
    body = _make_mp_kernel(n, nt, te, vec is not None, wo_w is not None)
    return pl.pallas_call(
        body,
        out_shape=jax.ShapeDtypeStruct((n, out_c), F32),
    )(*args)


NREC = 512
NLIG = 64
NC = NREC + NLIG
CI = 128


def _make_cross_kernel(nt_rec, te):
    def body(hc_ref, x_ref, rsrc_ref, rdst_ref, lsrc_ref, ldst_ref,
             wms_ref, wmd_ref, trows_ref, wmr_ref, wat_ref, whh_ref,
             wha_ref, out_ref, a_ref, b_ref, numl_ref, denl_ref):
        hc = hc_ref[...]
        x = x_ref[...]
        wms = wms_ref[...]
        wmd = wmd_ref[...]
        trows = trows_ref[...]
        wmr = wmr_ref[...]
        wat = wat_ref[...]
        whh = whh_ref[...]
        wha = wha_ref[...]

        hcb = hc.astype(BF16)
        a = jnp.dot(hcb, wms.astype(BF16), preferred_element_type=F32)
        b = jnp.dot(hcb, wmd.astype(BF16), preferred_element_type=F32)
        a_ref[...] = a
        b_ref[...] = b
        hw = jnp.dot(hcb, whh.astype(BF16), preferred_element_type=F32)
        c = a.shape[1]
        trowsb = trows.astype(BF16).astype(F32)
        t0 = trowsb[0:1, :]
        t1 = trowsb[1:2, :]
        t2 = trowsb[2:3, :]
        wmrb = wmr.astype(BF16).astype(F32)
        watb = wat.astype(BF16).astype(F32)

        ahi, alo = _split(a)
        bhi, blo = _split(b)
        xhi, xlo = _split(x)

        def sp_tile(srow, drow, tvec, num, den):
            ots = _onehot_t(srow, NC, srow.shape[1])
            otd = _onehot_t(drow, NC, drow.shape[1])
            xs = _gather_pair(ots, xhi, xlo)
            xd = _gather_pair(otd, xhi, xlo)
            d = xd - xs
            r = jnp.sqrt(jnp.sum(d * d, axis=1, keepdims=True) + 1e-12)
            rb = r.astype(BF16).astype(F32)
            m = _gather_pair(ots, ahi, alo) + _gather_pair(otd, bhi, blo) \
                + tvec + rb * wmrb
            m = jnp.maximum(m, 0.0)
            mb = m.astype(BF16).astype(F32)
            tl = jnp.exp(jnp.sum(mb * watb, axis=1, keepdims=True))
            return num + _scat(otd, tl * m), den + _scat(otd, tl)

        def rec_tile(t, carry):
            num, den = carry
            return sp_tile(rsrc_ref[pl.ds(t, 1), :], rdst_ref[pl.ds(t, 1), :],
                           t0, num, den)

        num0 = jnp.zeros((NC, c), F32)
        den0 = jnp.zeros((NC, 1), F32)
        num, den = jax.lax.fori_loop(0, nt_rec, rec_tile, (num0, den0))
        num, den = sp_tile(lsrc_ref[0:1, :], ldst_ref[0:1, :], t1, num, den)

        xl = x[NREC:NC, :]
        al = a[NREC:NC, :]
        bl = b[NREC:NC, :]
        bl_t = jnp.tile(bl, (CI, 1))
        xl_t = jnp.tile(xl, (CI, 1))
        al_rep = jnp.repeat(al, CI, axis=0)
        xl_rep = jnp.repeat(xl, CI, axis=0)

        def chunk(ci, carry):
            num_l, den_l = carry
            arc = a_ref[pl.ds(ci * CI, CI), :]
            brc = b_ref[pl.ds(ci * CI, CI), :]
            xrc = x_ref[pl.ds(ci * CI, CI), :]
            ar_rep = jnp.repeat(arc, NLIG, axis=0)
            xr_rep = jnp.repeat(xrc, NLIG, axis=0)
            d1 = xl_t - xr_rep
            r1 = jnp.sqrt(jnp.sum(d1 * d1, axis=1, keepdims=True) + 1e-12)
            r1b = r1.astype(BF16).astype(F32)
            m1 = jnp.maximum(ar_rep + bl_t + t2 + r1b * wmrb, 0.0)
            m1b = m1.astype(BF16).astype(F32)
            tl1 = jnp.exp(jnp.sum(m1b * watb, axis=1, keepdims=True))
            num_l = num_l + jnp.sum((tl1 * m1).reshape(CI, NLIG, c), axis=0)
            den_l = den_l + jnp.sum(tl1.reshape(CI, NLIG, 1), axis=0)
            br_t = jnp.tile(brc, (NLIG, 1))
            xr_t = jnp.tile(xrc, (NLIG, 1))
            d2 = xr_t - xl_rep
            r2 = jnp.sqrt(jnp.sum(d2 * d2, axis=1, keepdims=True) + 1e-12)
            r2b = r2.astype(BF16).astype(F32)
            m2 = jnp.maximum(al_rep + br_t + t2 + r2b * wmrb, 0.0)
            m2b = m2.astype(BF16).astype(F32)
            tl2 = jnp.exp(jnp.sum(m2b * watb, axis=1, keepdims=True))
            numl_ref[pl.ds(ci * CI, CI), :] = jnp.sum(
                (tl2 * m2).reshape(NLIG, CI, c), axis=0)
            denl_ref[pl.ds(ci * CI, CI), :] = jnp.sum(
                tl2.reshape(NLIG, CI, 1), axis=0)
            return num_l, den_l

        numl0 = jnp.zeros((NLIG, c), F32)
        denl0 = jnp.zeros((NLIG, 1), F32)
        num_l, den_l = jax.lax.fori_loop(0, NREC // CI, chunk, (numl0, denl0))
        num = num + jnp.concatenate([numl_ref[...], num_l], axis=0)
        den = den + jnp.concatenate([denl_ref[...], den_l], axis=0)

        agg = num / (den + 1e-9)
        out_ref[...] = jnp.maximum(
            hw + jnp.dot(agg.astype(BF16), wha.astype(BF16),
                         preferred_element_type=F32), 0.0)

    return body


def _cross_layer(hc, x, rsrc2, rdst2, lsrc2, ldst2, wms, wmd, trows, wmr,
                 wat, whh, wha):
    nt_rec, te = rsrc2.shape
    c = wha.shape[1]
    body = _make_cross_kernel(nt_rec, te)
    return pl.pallas_call(
        body,
        out_shape=jax.ShapeDtypeStruct((NC, c), F32),
        scratch_shapes=[
            pltpu.VMEM((NC, c), F32),
            pltpu.VMEM((NC, c), F32),
            pltpu.VMEM((NREC, c), F32),
            pltpu.VMEM((NREC, 1), F32),
        ],
    )(hc, x, rsrc2, rdst2, lsrc2, ldst2, wms, wmd, trows, wmr, wat, whh,
      wha)


def _make_final_kernel(te):
    def body(hc_ref, x_ref, lsrc_ref, ldst_ref, wfs_ref, wfd_ref, trows_ref,
             wfr_ref, wdft_ref, out_ref, a_ref, b_ref, upd_ref):
        hc = hc_ref[...]
        x = x_ref[...]
        wfs = wfs_ref[...]
        wfd = wfd_ref[...]
        trows = trows_ref[...]
        wfr = wfr_ref[...]
        wdft = wdft_ref[...]

        hcb = hc.astype(BF16)
        a = jnp.dot(hcb, wfs.astype(BF16), preferred_element_type=F32)
        b = jnp.dot(hcb, wfd.astype(BF16), preferred_element_type=F32)
        b_ref[...] = b
        c = a.shape[1]
        trowsb = trows.astype(BF16).astype(F32)
        t1 = trowsb[1:2, :]
        t2 = trowsb[2:3, :]
        wfrb = wfr.astype(BF16).astype(F32)
        wdftb = wdft.astype(BF16).astype(F32)

        srow = lsrc_ref[0:1, :]
        drow = ldst_ref[0:1, :]
        ots = _onehot_t(srow, NC, te)
        otd = _onehot_t(drow, NC, te)
        xs = _gather(ots, x)
        xd = _gather(otd, x)
        d = xd - xs
        r = jnp.sqrt(jnp.sum(d * d, axis=1, keepdims=True) + 1e-12)
        rb = r.astype(BF16).astype(F32)
        f = jnp.maximum(
            _gather(ots, a) + _gather(otd, b) + t1 + rb * wfrb, 0.0)
        fb = f.astype(BF16).astype(F32)
        s = jnp.sum(fb * wdftb, axis=1, keepdims=True)
        su = s * (d / (r + 1e-9))
        otd64 = _onehot_t(drow - NREC, NLIG, te)
        upd_ref[...] = _scat(otd64, su)

        xl = x[NREC:NC, :]
        bl = b[NREC:NC, :]
        bl_t = jnp.tile(bl, (CI, 1))
        xl_t = jnp.tile(xl, (CI, 1))
        a_ref[...] = a

        def chunk(ci, upd_acc):
            arc = a_ref[pl.ds(ci * CI, CI), :]
            xrc = x_ref[pl.ds(ci * CI, CI), :]
            ar_rep = jnp.repeat(arc, NLIG, axis=0)
            xr_rep = jnp.repeat(xrc, NLIG, axis=0)
            d1 = xl_t - xr_rep
            r1 = jnp.sqrt(jnp.sum(d1 * d1, axis=1, keepdims=True) + 1e-12)
            r1b = r1.astype(BF16).astype(F32)
            f1 = jnp.maximum(ar_rep + bl_t + t2 + r1b * wfrb, 0.0)
            f1b = f1.astype(BF16).astype(F32)
            s1 = jnp.sum(f1b * wdftb, axis=1, keepdims=True)
            su1 = s1 * (d1 / (r1 + 1e-9))
            return upd_acc + jnp.sum(su1.reshape(CI, NLIG, 3), axis=0)

        upd_d = jax.lax.fori_loop(0, NREC // CI, chunk,
                                  jnp.zeros((NLIG, 3), F32))
        out_ref[...] = x[NREC:NC, :] + upd_ref[...] + upd_d

    return body


def _final_layer(hc, x, lsrc2, ldst2, wfs, wfd, trows, wfr, wdft):
    te = lsrc2.shape[1]
    return pl.pallas_call(
        _make_final_kernel(te),
        out_shape=jax.ShapeDtypeStruct((NLIG, 3), F32),
        scratch_shapes=[
            pltpu.VMEM((NC, wfd.shape[1]), F32),
            pltpu.VMEM((NC, wfd.shape[1]), F32),
            pltpu.VMEM((NLIG, 3), F32),
        ],
    )(hc, x, lsrc2, ldst2, wfs, wfd, trows, wfr, wdft)


def kernel(rec_x, rec_f, rec_vec, rec_edge_index, rec_edge_w, lig_x, lig_f,
           lig_edge_index, lig_edge_w, rWm0, rWa0, rWh0, rWm1, rWa1, rWh1,
           rWo, lWm0, lWa0, lWh0, lWm1, lWa1, lWh1, lWo, cWm0, cWa0, cWh0,
           cWm1, cWa1, cWh1, cWmf, cWdf):
    te = 2048
    rs = rec_edge_index[0].reshape(-1, te)
    rd = rec_edge_index[1].reshape(-1, te)
    ls = lig_edge_index[0].reshape(1, -1)
    ld = lig_edge_index[1].reshape(1, -1)
    vec = rec_vec[:, 0, :]

    h = _mp_layer(rec_f, vec, rec_x, rs, rd, rec_edge_w,
                  rWm0[0:129], rWm0[129:258], rWm0[258:262], rWm0[262:263],
                  rWa0.T, rWh0[0:129], rWh0[129:257], None)
    h_rec = _mp_layer(h, None, rec_x, rs, rd, rec_edge_w,
                      rWm1[0:128], rWm1[128:256], rWm1[256:260],
                      rWm1[260:261], rWa1.T, rWh1[0:128], rWh1[128:256], rWo)

    h = _mp_layer(lig_f, None, lig_x, ls, ld, lig_edge_w,
                  lWm0[0:64], lWm0[64:128], lWm0[128:132], lWm0[132:133],
                  lWa0.T, lWh0[0:64], lWh0[64:128], None)
    h_lig = _mp_layer(h, None, lig_x, ls, ld, lig_edge_w,
                      lWm1[0:64], lWm1[64:128], lWm1[128:132], lWm1[132:133],
                      lWa1.T, lWh1[0:64], lWh1[64:128], lWo)

    hc = jnp.concatenate([h_rec, h_lig], axis=0)
    x = jnp.concatenate([rec_x, lig_x], axis=0)
    lss = ls + NREC
    lds = ld + NREC
    hc = _cross_layer(hc, x, rs, rd, lss, lds,
                      cWm0[0:128], cWm0[128:256], cWm0[256:259],
                      cWm0[259:260], cWa0.T, cWh0[0:128], cWh0[128:256])
    hc = _cross_layer(hc, x, rs, rd, lss, lds,
                      cWm1[0:128], cWm1[128:256], cWm1[256:259],
                      cWm1[259:260], cWa1.T, cWh1[0:128], cWh1[128:256])

    lig_new = _final_layer(hc, x, lss, lds,
                           cWmf[0:128], cWmf[128:256], cWmf[256:259],
                           cWmf[259:260], cWdf.T)
    return jnp.stack([lig_x, lig_new])[None, :]
